# Initial kernel scaffold; baseline (speedup 1.0000x reference)
import jax
import jax.numpy as jnp
from jax.experimental import pallas as pl


def kernel(logits, uniform):
    raise NotImplementedError("write your pallas kernel here")


# speedup vs baseline: 1.7163x; 1.7163x over previous
import jax
import jax.numpy as jnp
from jax.experimental import pallas as pl

_TAU = 0.3


def _body(logits_ref, u_ref, out_ref):
    u = u_ref[...]
    lg = logits_ref[...]
    g = -jnp.log(-jnp.log(u))
    x = (g + lg[:, None, :]) / _TAU
    m = jnp.max(x, axis=2, keepdims=True)
    e = jnp.exp(x - m)
    s = jnp.sum(e, axis=2, keepdims=True)
    p = e / s
    out_ref[...] = jnp.max(p, axis=1)


_BB = 8


def kernel(logits, uniform):
    B, D = logits.shape
    K = uniform.shape[1]
    return pl.pallas_call(
        _body,
        grid=(B // _BB,),
        in_specs=[
            pl.BlockSpec((_BB, D), lambda b: (b, 0)),
            pl.BlockSpec((_BB, K, D), lambda b: (b, 0, 0)),
        ],
        out_specs=pl.BlockSpec((_BB, D), lambda b: (b, 0)),
        out_shape=jax.ShapeDtypeStruct((B, D), jnp.float32),
    )(logits, uniform)
